# baseline (speedup 1.0000x reference)
import jax
import jax.numpy as jnp
from jax.experimental import pallas as pl

N_NODES = 512
N_EDGES = 2048
HID = 128
NH = 8
DH = 16
NL = 2
NS = 32
NODE_DIM = 43
EDGE_DIM = 16
QT = 512
NEG = -1e9
SCALE = 1.0 / (DH ** 0.5)
F32 = jnp.float32


def _mm(a, b):
    return jax.lax.dot_general(a, b, (((1,), (0,)), ((), ())),
                               preferred_element_type=F32)


def _mm_nt(a, b):
    return jax.lax.dot_general(a, b, (((1,), (1,)), ((), ())),
                               preferred_element_type=F32)


def _mm_hi(a, b):
    return jax.lax.dot_general(a, b, (((1,), (0,)), ((), ())),
                               preferred_element_type=F32,
                               precision=jax.lax.Precision.HIGHEST)


def _silu(x):
    return x * jax.nn.sigmoid(x)


def _ln(x, g, b):
    m = jnp.mean(x, axis=-1, keepdims=True)
    v = jnp.mean((x - m) ** 2, axis=-1, keepdims=True)
    return (x - m) / jnp.sqrt(v + 1e-5) * g + b


def _softmax(x):
    m = jnp.max(x, axis=-1, keepdims=True)
    e = jnp.exp(x - m)
    return e / jnp.sum(e, axis=-1, keepdims=True)


def _fused(*refs):
    out_ref = refs[-1]
    it = iter(refs[:-1])

    def nxt():
        return next(it)[...]

    nf = nxt()
    ea = nxt()
    src = nxt()
    dst = nxt()
    dstr = nxt()
    bidc = nxt()
    bidr = nxt()

    e1a = nxt(); e1b = nxt(); e1c = nxt(); b1 = nxt()
    e2w = nxt(); b2 = nxt(); e3w = nxt(); b3 = nxt()

    layers = []
    for _ in range(NL):
        layers.append(dict(
            ln1g=nxt(), ln1b=nxt(),
            wq=nxt(), bq=nxt(), wk=nxt(), bk=nxt(), wv=nxt(), bv=nxt(),
            wo=nxt(), bo=nxt(), ln2g=nxt(), ln2b=nxt(),
            f1=nxt(), f1b=nxt(), f2=nxt(), f2b=nxt()))

    seeds = nxt()
    wqp = nxt(); bqp = nxt(); wkp = nxt(); bkp = nxt(); wvp = nxt(); bvp = nxt()
    woutT = nxt(); bout = nxt()
    wfm1 = nxt(); bfm1 = nxt(); wfm2 = nxt(); bfm2 = nxt()
    w1o = nxt(); b1o = nxt(); w2o = nxt(); b2o = nxt()

    iota_e = jax.lax.broadcasted_iota(jnp.int32, (N_EDGES, N_NODES), 1)
    oh_src = (iota_e == src).astype(F32)
    oh_dst = (iota_e == dst).astype(F32)
    nfs = _mm_hi(oh_src, nf)
    nfd = _mm_hi(oh_dst, nf)
    eblk_c = _mm_hi(oh_dst, bidc)
    iota_n = jax.lax.broadcasted_iota(jnp.int32, (N_NODES, N_EDGES), 0)
    oh_dst_t = (iota_n == dstr).astype(F32)
    eblk_r = _mm_hi(bidr, oh_dst_t)

    h = _silu(_mm(nfs, e1a) + _mm(nfd, e1b) + _mm(ea, e1c) + b1)
    h = _silu(_mm(h, e2w) + b2)
    ef = _mm(h, e3w) + b3

    eblk_ci = eblk_c.astype(jnp.int32)
    eblk_ri = eblk_r.astype(jnp.int32)
    iota_e8 = jax.lax.broadcasted_iota(jnp.int32, (N_EDGES, 8), 1)
    oh_blk = (iota_e8 == eblk_ci).astype(F32)
    iota_8e = jax.lax.broadcasted_iota(jnp.int32, (8, N_EDGES), 0)
    oh_blk_t = (iota_8e == eblk_ri).astype(F32)

    for lp in layers:
        hN = _ln(ef, lp["ln1g"], lp["ln1b"])
        khs, vhs = [], []
        for hd in range(NH):
            wk = lp["wk"][hd * DH:(hd + 1) * DH, :]
            wv = lp["wv"][hd * DH:(hd + 1) * DH, :]
            khs.append(_mm_nt(hN, wk) + lp["bk"][hd:hd + 1, :])
            vhs.append(_mm_nt(hN, wv) + lp["bv"][hd:hd + 1, :])
        tiles = []
        for t in range(N_EDGES // QT):
            sl = slice(t * QT, (t + 1) * QT)
            oh_q = oh_blk[sl, :]
            acc_t = jnp.zeros((QT, HID), F32)
            for hd in range(NH):
                wq = lp["wq"][hd * DH:(hd + 1) * DH, :]
                wo = lp["wo"][hd * DH:(hd + 1) * DH, :]
                qh = (_mm_nt(hN[sl, :], wq) + lp["bq"][hd:hd + 1, :]) * SCALE
                s = _mm_nt(qh, khs[hd])
                m = jnp.max(s, axis=-1, keepdims=True)
                e = jnp.exp(s - m)
                d_b = _mm(e, oh_blk)
                d1 = jnp.sum(d_b * oh_q, axis=-1, keepdims=True)
                dall = jnp.sum(d_b, axis=-1, keepdims=True)
                d2 = dall - d1
                r2 = jnp.where(d2 > 0.0, 1.0 / d2, 0.0)
                extra = jnp.where(d2 > 0.0, 0.0, 1.0 / dall)
                fac = (r2 + extra) + oh_q * (1.0 / d1 - r2)
                w = e * _mm(fac, oh_blk_t)
                o = _mm(w, vhs[hd])
                acc_t = acc_t + _mm(o, wo)
            tiles.append(acc_t)
        ef = ef + jnp.concatenate(tiles, axis=0) + lp["bo"]
        h2 = _ln(ef, lp["ln2g"], lp["ln2b"])
        ef = ef + _mm(_silu(_mm(h2, lp["f1"]) + lp["f1b"]), lp["f2"]) + lp["f2b"]

    gp = jnp.zeros((NS, HID), F32)
    for hd in range(NH):
        wq = wqp[hd * DH:(hd + 1) * DH, :]
        wk = wkp[hd * DH:(hd + 1) * DH, :]
        wv = wvp[hd * DH:(hd + 1) * DH, :]
        qh = _mm_nt(seeds, wq) + bqp[hd:hd + 1, :]
        kh = _mm_nt(ef, wk) + bkp[hd:hd + 1, :]
        vh = _mm_nt(ef, wv) + bvp[hd:hd + 1, :]
        w = _softmax(_mm_nt(qh, kh) * SCALE)
        ath = _mm(w, vh)
        gp = gp + _mm(ath, woutT[hd * DH:(hd + 1) * DH, :])
    G = gp + bout

    g1 = bfm1
    for s_i in range(NS):
        g1 = g1 + _mm(G[s_i:s_i + 1, :], wfm1[s_i * HID:(s_i + 1) * HID, :])
    g2 = _mm(_silu(g1), wfm2) + bfm2
    out = _mm(_silu(_mm(g2, w1o) + b1o), w2o) + b2o
    out_ref[...] = out


def kernel(node_features, node_coords, edge_index, edge_attr, block_ids, params):
    del node_coords
    src = edge_index[0].astype(jnp.int32).reshape(N_EDGES, 1)
    dst = edge_index[1].astype(jnp.int32).reshape(N_EDGES, 1)
    dstr = dst.reshape(1, N_EDGES)
    bidc = block_ids.astype(F32).reshape(N_NODES, 1)
    bidr = bidc.reshape(1, N_NODES)

    em = params["edge_mlp"]
    W1 = em[0]["W"]
    args = [
        node_features.astype(F32), edge_attr.astype(F32),
        src, dst, dstr, bidc, bidr,
        W1[:, :NODE_DIM].T, W1[:, NODE_DIM:2 * NODE_DIM].T,
        W1[:, 2 * NODE_DIM:].T, em[0]["b"][None, :],
        em[1]["W"].T, em[1]["b"][None, :],
        em[2]["W"].T, em[2]["b"][None, :],
    ]
    for bp in params["blocks"]:
        args += [
            bp["ln1"]["g"][None, :], bp["ln1"]["b"][None, :],
            bp["q"]["W"], bp["q"]["b"].reshape(NH, DH),
            bp["k"]["W"], bp["k"]["b"].reshape(NH, DH),
            bp["v"]["W"], bp["v"]["b"].reshape(NH, DH),
            bp["o"]["W"].T, bp["o"]["b"][None, :],
            bp["ln2"]["g"][None, :], bp["ln2"]["b"][None, :],
            bp["ff1"]["W"].T, bp["ff1"]["b"][None, :],
            bp["ff2"]["W"].T, bp["ff2"]["b"][None, :],
        ]
    pp = params["pool"]
    args += [
        pp["seeds"],
        pp["q"]["W"], pp["q"]["b"].reshape(NH, DH),
        pp["k"]["W"], pp["k"]["b"].reshape(NH, DH),
        pp["v"]["W"], pp["v"]["b"].reshape(NH, DH),
        pp["out"]["W"].T, pp["out"]["b"][None, :],
        pp["fm1"]["W"].T, pp["fm1"]["b"][None, :],
        pp["fm2"]["W"].T, pp["fm2"]["b"][None, :],
    ]
    args += [
        params["out1"]["W"].T, params["out1"]["b"][None, :],
        params["out2"]["W"].T, params["out2"]["b"][None, :],
    ]
    return pl.pallas_call(
        _fused,
        out_shape=jax.ShapeDtypeStruct((1, 1), F32),
    )(*args)


# speedup vs baseline: 1.2789x; 1.2789x over previous
import jax
import jax.numpy as jnp
from jax.experimental import pallas as pl

N_NODES = 512
N_EDGES = 2048
HID = 128
NH = 8
DH = 16
NL = 2
NS = 32
NODE_DIM = 43
EDGE_DIM = 16
QT = 512
NEG = -1e9
SCALE = 1.0 / (DH ** 0.5)
F32 = jnp.float32
BF16 = jnp.bfloat16


def _mm(a, b):
    return jax.lax.dot_general(a, b, (((1,), (0,)), ((), ())),
                               preferred_element_type=F32)


def _mm_nt(a, b):
    return jax.lax.dot_general(a, b, (((1,), (1,)), ((), ())),
                               preferred_element_type=F32)


def _mm_hi(a, b):
    return jax.lax.dot_general(a, b, (((1,), (0,)), ((), ())),
                               preferred_element_type=F32,
                               precision=jax.lax.Precision.HIGHEST)


def _silu(x):
    return x * jax.nn.sigmoid(x)


def _ln(x, g, b):
    m = jnp.mean(x, axis=-1, keepdims=True)
    v = jnp.mean((x - m) ** 2, axis=-1, keepdims=True)
    return (x - m) / jnp.sqrt(v + 1e-5) * g + b


def _softmax(x):
    m = jnp.max(x, axis=-1, keepdims=True)
    e = jnp.exp(x - m)
    return e / jnp.sum(e, axis=-1, keepdims=True)


def _fused(*refs):
    out_ref = refs[-1]
    it = iter(refs[:-1])

    def nxt():
        return next(it)[...]

    nf = nxt()
    ea = nxt()
    src = nxt()
    dst = nxt()
    dstr = nxt()
    bidc = nxt()
    bidr = nxt()

    e1a = nxt(); e1b = nxt(); e1c = nxt(); b1 = nxt()
    e2w = nxt(); b2 = nxt(); e3w = nxt(); b3 = nxt()

    layers = []
    for _ in range(NL):
        layers.append(dict(
            ln1g=nxt(), ln1b=nxt(),
            wq=nxt(), bq=nxt(), wk=nxt(), bk=nxt(), wv=nxt(), bv=nxt(),
            wo=nxt(), bo=nxt(), ln2g=nxt(), ln2b=nxt(),
            f1=nxt(), f1b=nxt(), f2=nxt(), f2b=nxt()))

    seeds = nxt()
    wqp = nxt(); bqp = nxt(); wkp = nxt(); bkp = nxt(); wvp = nxt(); bvp = nxt()
    woutT = nxt(); bout = nxt()
    wfm1 = nxt(); bfm1 = nxt(); wfm2 = nxt(); bfm2 = nxt()
    w1o = nxt(); b1o = nxt(); w2o = nxt(); b2o = nxt()

    iota_e = jax.lax.broadcasted_iota(jnp.int32, (N_EDGES, N_NODES), 1)
    oh_src = (iota_e == src).astype(F32)
    oh_dst = (iota_e == dst).astype(F32)
    nfs = _mm_hi(oh_src, nf)
    nfd = _mm_hi(oh_dst, nf)
    eblk_c = _mm_hi(oh_dst, bidc)
    iota_n = jax.lax.broadcasted_iota(jnp.int32, (N_NODES, N_EDGES), 0)
    oh_dst_t = (iota_n == dstr).astype(F32)
    eblk_r = _mm_hi(bidr, oh_dst_t)

    h = _silu(_mm(nfs, e1a) + _mm(nfd, e1b) + _mm(ea, e1c) + b1)
    h = _silu(_mm(h, e2w) + b2)
    ef = _mm(h, e3w) + b3

    eblk_ci = eblk_c.astype(jnp.int32)
    eblk_ri = eblk_r.astype(jnp.int32)
    iota_e8 = jax.lax.broadcasted_iota(jnp.int32, (N_EDGES, 8), 1)
    oh_blk = (iota_e8 == eblk_ci).astype(F32)
    iota_8e = jax.lax.broadcasted_iota(jnp.int32, (8, N_EDGES), 0)
    oh_blk_t = (iota_8e == eblk_ri).astype(F32)

    for lp in layers:
        hN = _ln(ef, lp["ln1g"], lp["ln1b"])
        khs, vhs = [], []
        for hd in range(NH):
            wk = lp["wk"][hd * DH:(hd + 1) * DH, :]
            wv = lp["wv"][hd * DH:(hd + 1) * DH, :]
            khs.append((_mm_nt(hN, wk) + lp["bk"][hd:hd + 1, :]).astype(BF16))
            vhs.append((_mm_nt(hN, wv) + lp["bv"][hd:hd + 1, :]).astype(BF16))
        tiles = []
        for t in range(N_EDGES // QT):
            sl = slice(t * QT, (t + 1) * QT)
            intra = eblk_c[sl, :] == eblk_r
            acc_t = jnp.zeros((QT, HID), F32)
            for hd in range(NH):
                wq = lp["wq"][hd * DH:(hd + 1) * DH, :]
                wo = lp["wo"][hd * DH:(hd + 1) * DH, :]
                qh = ((_mm_nt(hN[sl, :], wq) + lp["bq"][hd:hd + 1, :])
                      * SCALE).astype(BF16)
                s = _mm_nt(qh, khs[hd])
                m = jnp.max(s, axis=-1, keepdims=True)
                e = jnp.exp(s - m)
                d1 = jnp.sum(jnp.where(intra, e, 0.0), axis=-1, keepdims=True)
                dall = jnp.sum(e, axis=-1, keepdims=True)
                d2 = dall - d1
                recip2 = jnp.where(d2 > 0.0, 1.0 / d2, 0.0)
                extra = jnp.where(d2 > 0.0, 0.0, 1.0 / dall)
                w = (e * (jnp.where(intra, 1.0 / d1, recip2) + extra)).astype(BF16)
                o = _mm(w, vhs[hd])
                acc_t = acc_t + _mm(o, wo)
            tiles.append(acc_t)
        ef = ef + jnp.concatenate(tiles, axis=0) + lp["bo"]
        h2 = _ln(ef, lp["ln2g"], lp["ln2b"])
        ef = ef + _mm(_silu(_mm(h2, lp["f1"]) + lp["f1b"]), lp["f2"]) + lp["f2b"]

    gp = jnp.zeros((NS, HID), F32)
    for hd in range(NH):
        wq = wqp[hd * DH:(hd + 1) * DH, :]
        wk = wkp[hd * DH:(hd + 1) * DH, :]
        wv = wvp[hd * DH:(hd + 1) * DH, :]
        qh = _mm_nt(seeds, wq) + bqp[hd:hd + 1, :]
        kh = _mm_nt(ef, wk) + bkp[hd:hd + 1, :]
        vh = _mm_nt(ef, wv) + bvp[hd:hd + 1, :]
        w = _softmax(_mm_nt(qh, kh) * SCALE)
        ath = _mm(w, vh)
        gp = gp + _mm(ath, woutT[hd * DH:(hd + 1) * DH, :])
    G = gp + bout

    g1 = bfm1
    for s_i in range(NS):
        g1 = g1 + _mm(G[s_i:s_i + 1, :], wfm1[s_i * HID:(s_i + 1) * HID, :])
    g2 = _mm(_silu(g1), wfm2) + bfm2
    out = _mm(_silu(_mm(g2, w1o) + b1o), w2o) + b2o
    out_ref[...] = out


def kernel(node_features, node_coords, edge_index, edge_attr, block_ids, params):
    del node_coords
    src = edge_index[0].astype(jnp.int32).reshape(N_EDGES, 1)
    dst = edge_index[1].astype(jnp.int32).reshape(N_EDGES, 1)
    dstr = dst.reshape(1, N_EDGES)
    bidc = block_ids.astype(F32).reshape(N_NODES, 1)
    bidr = bidc.reshape(1, N_NODES)

    em = params["edge_mlp"]
    W1 = em[0]["W"]
    args = [
        node_features.astype(F32), edge_attr.astype(F32),
        src, dst, dstr, bidc, bidr,
        W1[:, :NODE_DIM].T, W1[:, NODE_DIM:2 * NODE_DIM].T,
        W1[:, 2 * NODE_DIM:].T, em[0]["b"][None, :],
        em[1]["W"].T, em[1]["b"][None, :],
        em[2]["W"].T, em[2]["b"][None, :],
    ]
    for bp in params["blocks"]:
        args += [
            bp["ln1"]["g"][None, :], bp["ln1"]["b"][None, :],
            bp["q"]["W"], bp["q"]["b"].reshape(NH, DH),
            bp["k"]["W"], bp["k"]["b"].reshape(NH, DH),
            bp["v"]["W"], bp["v"]["b"].reshape(NH, DH),
            bp["o"]["W"].T, bp["o"]["b"][None, :],
            bp["ln2"]["g"][None, :], bp["ln2"]["b"][None, :],
            bp["ff1"]["W"].T, bp["ff1"]["b"][None, :],
            bp["ff2"]["W"].T, bp["ff2"]["b"][None, :],
        ]
    pp = params["pool"]
    args += [
        pp["seeds"],
        pp["q"]["W"], pp["q"]["b"].reshape(NH, DH),
        pp["k"]["W"], pp["k"]["b"].reshape(NH, DH),
        pp["v"]["W"], pp["v"]["b"].reshape(NH, DH),
        pp["out"]["W"].T, pp["out"]["b"][None, :],
        pp["fm1"]["W"].T, pp["fm1"]["b"][None, :],
        pp["fm2"]["W"].T, pp["fm2"]["b"][None, :],
    ]
    args += [
        params["out1"]["W"].T, params["out1"]["b"][None, :],
        params["out2"]["W"].T, params["out2"]["b"][None, :],
    ]
    return pl.pallas_call(
        _fused,
        out_shape=jax.ShapeDtypeStruct((1, 1), F32),
    )(*args)


# speedup vs baseline: 1.3356x; 1.0443x over previous
import functools

import jax
import jax.numpy as jnp
from jax import lax
from jax.experimental import pallas as pl
from jax.experimental.pallas import tpu as pltpu
from jax.experimental.pallas import tpu_sc as plsc

N_NODES = 512
N_EDGES = 2048
HID = 128
NH = 8
DH = 16
NL = 2
NS = 32
NODE_DIM = 43
EDGE_DIM = 16
QT = 512
NEG = -1e9
SCALE = 1.0 / (DH ** 0.5)
F32 = jnp.float32
BF16 = jnp.bfloat16


def _mm(a, b):
    return jax.lax.dot_general(a, b, (((1,), (0,)), ((), ())),
                               preferred_element_type=F32)


def _mm_nt(a, b):
    return jax.lax.dot_general(a, b, (((1,), (1,)), ((), ())),
                               preferred_element_type=F32)


def _mm_hi(a, b):
    return jax.lax.dot_general(a, b, (((1,), (0,)), ((), ())),
                               preferred_element_type=F32,
                               precision=jax.lax.Precision.HIGHEST)


def _silu(x):
    return x * jax.nn.sigmoid(x)


def _ln(x, g, b):
    m = jnp.mean(x, axis=-1, keepdims=True)
    v = jnp.mean((x - m) ** 2, axis=-1, keepdims=True)
    return (x - m) / jnp.sqrt(v + 1e-5) * g + b


def _softmax(x):
    m = jnp.max(x, axis=-1, keepdims=True)
    e = jnp.exp(x - m)
    return e / jnp.sum(e, axis=-1, keepdims=True)


GD = 128
NW = 32
EPW = N_EDGES // NW


def _sc_gather(table, src, dst):
    mesh = plsc.VectorSubcoreMesh(core_axis_name="c", subcore_axis_name="s")

    @functools.partial(
        pl.kernel, mesh=mesh,
        out_type=[jax.ShapeDtypeStruct((N_EDGES, GD), jnp.float32),
                  jax.ShapeDtypeStruct((N_EDGES, GD), jnp.float32)],
        scratch_types=[pltpu.VMEM((EPW,), jnp.int32),
                       pltpu.VMEM((EPW,), jnp.int32),
                       pltpu.VMEM((EPW, GD), jnp.float32),
                       pltpu.VMEM((EPW, GD), jnp.float32),
                       pltpu.SemaphoreType.DMA,
                       pltpu.SemaphoreType.DMA],
    )
    def k(table_hbm, src_hbm, dst_hbm, outs_hbm, outd_hbm,
          sidx_v, didx_v, srows_v, drows_v, sem_s, sem_d):
        wid = lax.axis_index("s") * 2 + lax.axis_index("c")
        base = wid * EPW
        pltpu.sync_copy(src_hbm.at[pl.ds(base, EPW)], sidx_v)
        pltpu.sync_copy(dst_hbm.at[pl.ds(base, EPW)], didx_v)
        cp_s = pltpu.async_copy(table_hbm.at[sidx_v], srows_v, sem_s)
        cp_d = pltpu.async_copy(table_hbm.at[didx_v], drows_v, sem_d)
        cp_s.wait()
        cp_d.wait()
        pltpu.sync_copy(srows_v, outs_hbm.at[pl.ds(base, EPW)])
        pltpu.sync_copy(drows_v, outd_hbm.at[pl.ds(base, EPW)])

    return k(table, src, dst)


def _fused(*refs):
    out_ref = refs[-1]
    it = iter(refs[:-1])

    def nxt():
        return next(it)[...]

    nfs = nxt()
    nfd = nxt()
    ea = nxt()
    eblk_c = nxt()
    eblk_r = nxt()

    e1a = nxt(); e1b = nxt(); e1c = nxt(); b1 = nxt()
    e2w = nxt(); b2 = nxt(); e3w = nxt(); b3 = nxt()

    layers = []
    for _ in range(NL):
        layers.append(dict(
            ln1g=nxt(), ln1b=nxt(),
            wq=nxt(), bq=nxt(), wk=nxt(), bk=nxt(), wv=nxt(), bv=nxt(),
            wo=nxt(), bo=nxt(), ln2g=nxt(), ln2b=nxt(),
            f1=nxt(), f1b=nxt(), f2=nxt(), f2b=nxt()))

    seeds = nxt()
    wqp = nxt(); bqp = nxt(); wkp = nxt(); bkp = nxt(); wvp = nxt(); bvp = nxt()
    woutT = nxt(); bout = nxt()
    wfm1 = nxt(); bfm1 = nxt(); wfm2 = nxt(); bfm2 = nxt()
    w1o = nxt(); b1o = nxt(); w2o = nxt(); b2o = nxt()

    h = _silu(_mm(nfs, e1a) + _mm(nfd, e1b) + _mm(ea, e1c) + b1)
    h = _silu(_mm(h, e2w) + b2)
    ef = _mm(h, e3w) + b3


    for lp in layers:
        hN = _ln(ef, lp["ln1g"], lp["ln1b"])
        khs, vhs = [], []
        for hd in range(NH):
            wk = lp["wk"][hd * DH:(hd + 1) * DH, :]
            wv = lp["wv"][hd * DH:(hd + 1) * DH, :]
            khs.append((_mm_nt(hN, wk) + lp["bk"][hd:hd + 1, :]).astype(BF16))
            vhs.append((_mm_nt(hN, wv) + lp["bv"][hd:hd + 1, :]).astype(BF16))
        tiles = []
        for t in range(N_EDGES // QT):
            sl = slice(t * QT, (t + 1) * QT)
            intra = eblk_c[sl, :] == eblk_r
            acc_t = jnp.zeros((QT, HID), F32)
            for hd in range(NH):
                wq = lp["wq"][hd * DH:(hd + 1) * DH, :]
                wo = lp["wo"][hd * DH:(hd + 1) * DH, :]
                qh = ((_mm_nt(hN[sl, :], wq) + lp["bq"][hd:hd + 1, :])
                      * SCALE).astype(BF16)
                s = _mm_nt(qh, khs[hd])
                m = jnp.max(s, axis=-1, keepdims=True)
                e = jnp.exp(s - m)
                d1 = jnp.sum(jnp.where(intra, e, 0.0), axis=-1, keepdims=True)
                dall = jnp.sum(e, axis=-1, keepdims=True)
                d2 = dall - d1
                recip2 = jnp.where(d2 > 0.0, 1.0 / d2, 0.0)
                extra = jnp.where(d2 > 0.0, 0.0, 1.0 / dall)
                w = (e * (jnp.where(intra, 1.0 / d1, recip2) + extra)).astype(BF16)
                o = _mm(w, vhs[hd])
                acc_t = acc_t + _mm(o, wo)
            tiles.append(acc_t)
        ef = ef + jnp.concatenate(tiles, axis=0) + lp["bo"]
        h2 = _ln(ef, lp["ln2g"], lp["ln2b"])
        ef = ef + _mm(_silu(_mm(h2, lp["f1"]) + lp["f1b"]), lp["f2"]) + lp["f2b"]

    gp = jnp.zeros((NS, HID), F32)
    for hd in range(NH):
        wq = wqp[hd * DH:(hd + 1) * DH, :]
        wk = wkp[hd * DH:(hd + 1) * DH, :]
        wv = wvp[hd * DH:(hd + 1) * DH, :]
        qh = _mm_nt(seeds, wq) + bqp[hd:hd + 1, :]
        kh = _mm_nt(ef, wk) + bkp[hd:hd + 1, :]
        vh = _mm_nt(ef, wv) + bvp[hd:hd + 1, :]
        w = _softmax(_mm_nt(qh, kh) * SCALE)
        ath = _mm(w, vh)
        gp = gp + _mm(ath, woutT[hd * DH:(hd + 1) * DH, :])
    G = gp + bout

    g1 = bfm1
    for s_i in range(NS):
        g1 = g1 + _mm(G[s_i:s_i + 1, :], wfm1[s_i * HID:(s_i + 1) * HID, :])
    g2 = _mm(_silu(g1), wfm2) + bfm2
    out = _mm(_silu(_mm(g2, w1o) + b1o), w2o) + b2o
    out_ref[...] = out


def kernel(node_features, node_coords, edge_index, edge_attr, block_ids, params):
    del node_coords
    src = edge_index[0].astype(jnp.int32).reshape(N_EDGES)
    dst = edge_index[1].astype(jnp.int32).reshape(N_EDGES)

    table = jnp.concatenate(
        [node_features.astype(F32),
         block_ids.astype(F32).reshape(N_NODES, 1),
         jnp.zeros((N_NODES, GD - NODE_DIM - 1), F32)], axis=1)
    nfs, nfd = _sc_gather(table, src, dst)
    eblk_c = lax.slice(nfd, (0, NODE_DIM), (N_EDGES, NODE_DIM + 1))
    eblk_r = eblk_c.reshape(1, N_EDGES)

    em = params["edge_mlp"]
    W1 = em[0]["W"]

    def _padgd(w):
        wt = w.T
        return jnp.concatenate(
            [wt, jnp.zeros((GD - NODE_DIM, wt.shape[1]), F32)], axis=0)

    args = [
        nfs, nfd, edge_attr.astype(F32), eblk_c, eblk_r,
        _padgd(W1[:, :NODE_DIM]), _padgd(W1[:, NODE_DIM:2 * NODE_DIM]),
        W1[:, 2 * NODE_DIM:].T, em[0]["b"][None, :],
        em[1]["W"].T, em[1]["b"][None, :],
        em[2]["W"].T, em[2]["b"][None, :],
    ]
    for bp in params["blocks"]:
        args += [
            bp["ln1"]["g"][None, :], bp["ln1"]["b"][None, :],
            bp["q"]["W"], bp["q"]["b"].reshape(NH, DH),
            bp["k"]["W"], bp["k"]["b"].reshape(NH, DH),
            bp["v"]["W"], bp["v"]["b"].reshape(NH, DH),
            bp["o"]["W"].T, bp["o"]["b"][None, :],
            bp["ln2"]["g"][None, :], bp["ln2"]["b"][None, :],
            bp["ff1"]["W"].T, bp["ff1"]["b"][None, :],
            bp["ff2"]["W"].T, bp["ff2"]["b"][None, :],
        ]
    pp = params["pool"]
    args += [
        pp["seeds"],
        pp["q"]["W"], pp["q"]["b"].reshape(NH, DH),
        pp["k"]["W"], pp["k"]["b"].reshape(NH, DH),
        pp["v"]["W"], pp["v"]["b"].reshape(NH, DH),
        pp["out"]["W"].T, pp["out"]["b"][None, :],
        pp["fm1"]["W"].T, pp["fm1"]["b"][None, :],
        pp["fm2"]["W"].T, pp["fm2"]["b"][None, :],
    ]
    args += [
        params["out1"]["W"].T, params["out1"]["b"][None, :],
        params["out2"]["W"].T, params["out2"]["b"][None, :],
    ]
    return pl.pallas_call(
        _fused,
        out_shape=jax.ShapeDtypeStruct((1, 1), F32),
    )(*args)


# speedup vs baseline: 1.3706x; 1.0262x over previous
import functools

import jax
import jax.numpy as jnp
from jax import lax
from jax.experimental import pallas as pl
from jax.experimental.pallas import tpu as pltpu
from jax.experimental.pallas import tpu_sc as plsc

N_NODES = 512
N_EDGES = 2048
HID = 128
NH = 8
DH = 16
NL = 2
NS = 32
NODE_DIM = 43
EDGE_DIM = 16
QT = 512
NEG = -1e9
SCALE = 1.0 / (DH ** 0.5)
F32 = jnp.float32
BF16 = jnp.bfloat16


def _mm(a, b):
    return jax.lax.dot_general(a, b, (((1,), (0,)), ((), ())),
                               preferred_element_type=F32)


def _mm_nt(a, b):
    return jax.lax.dot_general(a, b, (((1,), (1,)), ((), ())),
                               preferred_element_type=F32)


def _mm_hi(a, b):
    return jax.lax.dot_general(a, b, (((1,), (0,)), ((), ())),
                               preferred_element_type=F32,
                               precision=jax.lax.Precision.HIGHEST)


def _silu(x):
    return x * jax.nn.sigmoid(x)


def _ln(x, g, b):
    m = jnp.mean(x, axis=-1, keepdims=True)
    v = jnp.mean((x - m) ** 2, axis=-1, keepdims=True)
    return (x - m) / jnp.sqrt(v + 1e-5) * g + b


def _softmax(x):
    m = jnp.max(x, axis=-1, keepdims=True)
    e = jnp.exp(x - m)
    return e / jnp.sum(e, axis=-1, keepdims=True)


GD = 128
NW = 32
EPW = N_EDGES // NW


def _sc_gather(table, src, dst):
    mesh = plsc.VectorSubcoreMesh(core_axis_name="c", subcore_axis_name="s")

    @functools.partial(
        pl.kernel, mesh=mesh,
        out_type=[jax.ShapeDtypeStruct((N_EDGES, GD), jnp.float32),
                  jax.ShapeDtypeStruct((N_EDGES, GD), jnp.float32)],
        scratch_types=[pltpu.VMEM((EPW,), jnp.int32),
                       pltpu.VMEM((EPW,), jnp.int32),
                       pltpu.VMEM((EPW, GD), jnp.float32),
                       pltpu.VMEM((EPW, GD), jnp.float32),
                       pltpu.SemaphoreType.DMA,
                       pltpu.SemaphoreType.DMA],
    )
    def k(table_hbm, src_hbm, dst_hbm, outs_hbm, outd_hbm,
          sidx_v, didx_v, srows_v, drows_v, sem_s, sem_d):
        wid = lax.axis_index("s") * 2 + lax.axis_index("c")
        base = wid * EPW
        pltpu.sync_copy(src_hbm.at[pl.ds(base, EPW)], sidx_v)
        pltpu.sync_copy(dst_hbm.at[pl.ds(base, EPW)], didx_v)
        cp_s = pltpu.async_copy(table_hbm.at[sidx_v], srows_v, sem_s)
        cp_d = pltpu.async_copy(table_hbm.at[didx_v], drows_v, sem_d)
        cp_s.wait()
        cp_d.wait()
        pltpu.sync_copy(srows_v, outs_hbm.at[pl.ds(base, EPW)])
        pltpu.sync_copy(drows_v, outd_hbm.at[pl.ds(base, EPW)])

    return k(table, src, dst)


def _fused(*refs):
    out_ref = refs[-1]
    it = iter(refs[:-1])

    def nxt():
        return next(it)[...]

    nfs = nxt()
    nfd = nxt()
    ea = nxt()
    eblk_c = nxt()
    eblk_r = nxt()

    e1a = nxt(); e1b = nxt(); e1c = nxt(); b1 = nxt()
    e2w = nxt(); b2 = nxt(); e3w = nxt(); b3 = nxt()

    layers = []
    for _ in range(NL):
        layers.append(dict(
            ln1g=nxt(), ln1b=nxt(),
            wq=nxt(), bq=nxt(), wk=nxt(), bk=nxt(), wv=nxt(), bv=nxt(),
            wo=nxt(), bo=nxt(), ln2g=nxt(), ln2b=nxt(),
            f1=nxt(), f1b=nxt(), f2=nxt(), f2b=nxt()))

    seeds = nxt()
    wqp = nxt(); bqp = nxt(); wkp = nxt(); bkp = nxt(); wvp = nxt(); bvp = nxt()
    woutT = nxt(); bout = nxt()
    wfm1 = nxt(); bfm1 = nxt(); wfm2 = nxt(); bfm2 = nxt()
    w1o = nxt(); b1o = nxt(); w2o = nxt(); b2o = nxt()

    h = _silu(_mm(nfs, e1a) + _mm(nfd, e1b) + _mm(ea, e1c) + b1)
    h = _silu(_mm(h, e2w) + b2)
    ef = _mm(h, e3w) + b3


    for lp in layers:
        hN = _ln(ef, lp["ln1g"], lp["ln1b"])
        khs, vhs = [], []
        for hd in range(NH):
            wk = lp["wk"][hd * DH:(hd + 1) * DH, :]
            wv = lp["wv"][hd * DH:(hd + 1) * DH, :]
            khs.append((_mm_nt(hN, wk) + lp["bk"][hd:hd + 1, :]).astype(BF16))
            vhs.append((_mm_nt(hN, wv) + lp["bv"][hd:hd + 1, :]).astype(BF16))
        tiles = []
        for t in range(N_EDGES // QT):
            sl = slice(t * QT, (t + 1) * QT)
            intra = eblk_c[sl, :] == eblk_r
            acc_t = jnp.zeros((QT, HID), F32)
            for hd in range(NH):
                wq = lp["wq"][hd * DH:(hd + 1) * DH, :]
                wo = lp["wo"][hd * DH:(hd + 1) * DH, :]
                qh = ((_mm_nt(hN[sl, :], wq) + lp["bq"][hd:hd + 1, :])
                      * SCALE).astype(BF16)
                s = _mm_nt(qh, khs[hd])
                m = jnp.max(s, axis=-1, keepdims=True)
                e = jnp.exp(s - m)
                d1 = jnp.sum(jnp.where(intra, e, 0.0), axis=-1, keepdims=True)
                dall = jnp.sum(e, axis=-1, keepdims=True)
                d2 = dall - d1
                extra = jnp.where(d2 > 0.0, 0.0, 1.0 / dall)
                r1x = 1.0 / d1 + extra
                r2x = jnp.where(d2 > 0.0, 1.0 / d2, 0.0) + extra
                w = (e * jnp.where(intra, r1x, r2x)).astype(BF16)
                o = _mm(w, vhs[hd])
                acc_t = acc_t + _mm(o, wo)
            tiles.append(acc_t)
        ef = ef + jnp.concatenate(tiles, axis=0) + lp["bo"]
        h2 = _ln(ef, lp["ln2g"], lp["ln2b"])
        ef = ef + _mm(_silu(_mm(h2, lp["f1"]) + lp["f1b"]), lp["f2"]) + lp["f2b"]

    gp = jnp.zeros((NS, HID), F32)
    for hd in range(NH):
        wq = wqp[hd * DH:(hd + 1) * DH, :]
        wk = wkp[hd * DH:(hd + 1) * DH, :]
        wv = wvp[hd * DH:(hd + 1) * DH, :]
        qh = _mm_nt(seeds, wq) + bqp[hd:hd + 1, :]
        kh = _mm_nt(ef, wk) + bkp[hd:hd + 1, :]
        vh = _mm_nt(ef, wv) + bvp[hd:hd + 1, :]
        w = _softmax(_mm_nt(qh, kh) * SCALE)
        ath = _mm(w, vh)
        gp = gp + _mm(ath, woutT[hd * DH:(hd + 1) * DH, :])
    G = gp + bout

    g1 = bfm1
    for s_i in range(NS):
        g1 = g1 + _mm(G[s_i:s_i + 1, :], wfm1[s_i * HID:(s_i + 1) * HID, :])
    g2 = _mm(_silu(g1), wfm2) + bfm2
    out = _mm(_silu(_mm(g2, w1o) + b1o), w2o) + b2o
    out_ref[...] = out


def kernel(node_features, node_coords, edge_index, edge_attr, block_ids, params):
    del node_coords
    src = edge_index[0].astype(jnp.int32).reshape(N_EDGES)
    dst = edge_index[1].astype(jnp.int32).reshape(N_EDGES)

    table = jnp.concatenate(
        [node_features.astype(F32),
         block_ids.astype(F32).reshape(N_NODES, 1),
         jnp.zeros((N_NODES, GD - NODE_DIM - 1), F32)], axis=1)
    nfs, nfd = _sc_gather(table, src, dst)
    eblk_c = lax.slice(nfd, (0, NODE_DIM), (N_EDGES, NODE_DIM + 1))
    eblk_r = eblk_c.reshape(1, N_EDGES)

    em = params["edge_mlp"]
    W1 = em[0]["W"]

    def _padgd(w):
        wt = w.T
        return jnp.concatenate(
            [wt, jnp.zeros((GD - NODE_DIM, wt.shape[1]), F32)], axis=0)

    args = [
        nfs, nfd, edge_attr.astype(F32), eblk_c, eblk_r,
        _padgd(W1[:, :NODE_DIM]), _padgd(W1[:, NODE_DIM:2 * NODE_DIM]),
        W1[:, 2 * NODE_DIM:].T, em[0]["b"][None, :],
        em[1]["W"].T, em[1]["b"][None, :],
        em[2]["W"].T, em[2]["b"][None, :],
    ]
    for bp in params["blocks"]:
        args += [
            bp["ln1"]["g"][None, :], bp["ln1"]["b"][None, :],
            bp["q"]["W"], bp["q"]["b"].reshape(NH, DH),
            bp["k"]["W"], bp["k"]["b"].reshape(NH, DH),
            bp["v"]["W"], bp["v"]["b"].reshape(NH, DH),
            bp["o"]["W"].T, bp["o"]["b"][None, :],
            bp["ln2"]["g"][None, :], bp["ln2"]["b"][None, :],
            bp["ff1"]["W"].T, bp["ff1"]["b"][None, :],
            bp["ff2"]["W"].T, bp["ff2"]["b"][None, :],
        ]
    pp = params["pool"]
    args += [
        pp["seeds"],
        pp["q"]["W"], pp["q"]["b"].reshape(NH, DH),
        pp["k"]["W"], pp["k"]["b"].reshape(NH, DH),
        pp["v"]["W"], pp["v"]["b"].reshape(NH, DH),
        pp["out"]["W"].T, pp["out"]["b"][None, :],
        pp["fm1"]["W"].T, pp["fm1"]["b"][None, :],
        pp["fm2"]["W"].T, pp["fm2"]["b"][None, :],
    ]
    args += [
        params["out1"]["W"].T, params["out1"]["b"][None, :],
        params["out2"]["W"].T, params["out2"]["b"][None, :],
    ]
    return pl.pallas_call(
        _fused,
        out_shape=jax.ShapeDtypeStruct((1, 1), F32),
    )(*args)
